# Initial kernel scaffold; baseline (speedup 1.0000x reference)
import jax
import jax.numpy as jnp
from jax.experimental import pallas as pl


def kernel(x, edge_index, edge_attr, epsilon, We, be, W1, b1, g1, bt1, W2, b2, g2, bt2):
    raise NotImplementedError("write your pallas kernel here")


# speedup vs baseline: 2.7417x; 2.7417x over previous
import functools

import jax
import jax.numpy as jnp
from jax import lax
from jax.experimental import pallas as pl
from jax.experimental.pallas import tpu as pltpu
from jax.experimental.pallas import tpu_sc as plsc

_NC = 2
_NS = 16


def _edge_feat_body(ea_ref, wet_ref, be_ref, out_ref):
    acc = jnp.dot(ea_ref[...], wet_ref[...], preferred_element_type=jnp.float32)
    out_ref[...] = jnp.maximum(acc + be_ref[...], 0.0)


@functools.partial(jax.jit, static_argnames=("be_block",))
def _edge_feat(edge_attr, wet, be_row, be_block=16000):
    E, DE = edge_attr.shape
    D = wet.shape[1]
    grid = E // be_block
    return pl.pallas_call(
        _edge_feat_body,
        grid=(grid,),
        in_specs=[
            pl.BlockSpec((be_block, DE), lambda i: (i, 0)),
            pl.BlockSpec((DE, D), lambda i: (0, 0)),
            pl.BlockSpec((1, D), lambda i: (0, 0)),
        ],
        out_specs=pl.BlockSpec((be_block, D), lambda i: (i, 0)),
        out_shape=jax.ShapeDtypeStruct((E, D), jnp.float32),
    )(edge_attr, wet, be_row)


def _make_sc_scatter(N, D, E, C=80):
    NW = _NC * _NS
    epw = E // NW
    nchunk = epw // C
    wchunk = 80
    nwc = N // wchunk

    mesh = plsc.VectorSubcoreMesh(core_axis_name="c", subcore_axis_name="s")

    @functools.partial(
        pl.kernel,
        out_type=jax.ShapeDtypeStruct((_NC * N, D), jnp.float32),
        mesh=mesh,
        scratch_types=[
            pltpu.VMEM((C,), jnp.int32),
            pltpu.VMEM((C,), jnp.int32),
            pltpu.VMEM((C, D), jnp.float32),
            pltpu.VMEM((C, D), jnp.float32),
            pltpu.VMEM_SHARED((N, D), jnp.float32),
            pltpu.SemaphoreType.DMA,
        ],
    )
    def sc_kernel(x_hbm, src_hbm, dst_hbm, feat_hbm, out_hbm,
                  src_v, dst_v, rows_v, feat_v, agg_sh, sem):
        c = lax.axis_index("c")
        s = lax.axis_index("s")

        def _zrow(i, _):
            for j in range(D // 16):
                rows_v[i, pl.ds(j * 16, 16)] = jnp.zeros((16,), jnp.float32)
            return 0
        lax.fori_loop(0, wchunk, _zrow, 0)

        def _zchunk(i, _):
            t = s + i * _NS

            @pl.when(t < nwc)
            def _():
                pltpu.sync_copy(rows_v, agg_sh.at[pl.ds(t * wchunk, wchunk)])
            return 0
        lax.fori_loop(0, (nwc + _NS - 1) // _NS, _zchunk, 0)
        plsc.subcore_barrier()

        base_e = (c * _NS + s) * epw

        def _chunk(g, _):
            eb = base_e + g * C
            pltpu.sync_copy(src_hbm.at[pl.ds(eb, C)], src_v)
            pltpu.sync_copy(dst_hbm.at[pl.ds(eb, C)], dst_v)
            pltpu.async_copy(x_hbm.at[src_v], rows_v, sem).wait()
            pltpu.sync_copy(feat_hbm.at[pl.ds(eb, C)], feat_v)
            pltpu.sync_copy(rows_v, agg_sh.at[dst_v], add=True)
            pltpu.sync_copy(feat_v, agg_sh.at[dst_v], add=True)
            return 0
        lax.fori_loop(0, nchunk, _chunk, 0)

        plsc.subcore_barrier()

        def _wchunk(i, _):
            t = s + i * _NS

            @pl.when(t < nwc)
            def _():
                pltpu.sync_copy(agg_sh.at[pl.ds(t * wchunk, wchunk)], rows_v)
                pltpu.sync_copy(
                    rows_v, out_hbm.at[pl.ds(c * N + t * wchunk, wchunk)])
            return 0
        lax.fori_loop(0, (nwc + _NS - 1) // _NS, _wchunk, 0)

    return sc_kernel


def _mlp_body(x_ref, agg_ref, eps_ref, w1t_ref, b1_ref, g1_ref, bt1_ref,
              w2t_ref, b2_ref, g2_ref, bt2_ref, out_ref):
    N = x_ref.shape[0]
    agg = agg_ref[:N, :] + agg_ref[N:, :]
    out = (1.0 + eps_ref[0]) * x_ref[...] + agg

    h = jnp.dot(out, w1t_ref[...], preferred_element_type=jnp.float32)
    h = h + b1_ref[...]
    mean = jnp.mean(h, axis=0, keepdims=True)
    var = jnp.mean((h - mean) ** 2, axis=0, keepdims=True)
    h = (h - mean) / jnp.sqrt(var + 1e-5) * g1_ref[...] + bt1_ref[...]
    h = jnp.maximum(h, 0.0)

    h = jnp.dot(h, w2t_ref[...], preferred_element_type=jnp.float32)
    h = h + b2_ref[...]
    mean = jnp.mean(h, axis=0, keepdims=True)
    var = jnp.mean((h - mean) ** 2, axis=0, keepdims=True)
    h = (h - mean) / jnp.sqrt(var + 1e-5) * g2_ref[...] + bt2_ref[...]
    out_ref[...] = jnp.maximum(h, 0.0)


def _combine_mlp(x, agg2, epsilon, w1t, b1, g1, bt1, w2t, b2, g2, bt2):
    N, D = x.shape
    vspec = pl.BlockSpec(memory_space=pltpu.VMEM)
    return pl.pallas_call(
        _mlp_body,
        in_specs=[vspec, vspec,
                  pl.BlockSpec(memory_space=pltpu.SMEM)] + [vspec] * 8,
        out_specs=vspec,
        out_shape=jax.ShapeDtypeStruct((N, D), jnp.float32),
    )(x, agg2, epsilon, w1t, b1, g1, bt1, w2t, b2, g2, bt2)


def kernel(x, edge_index, edge_attr, epsilon, We, be, W1, b1, g1, bt1,
           W2, b2, g2, bt2):
    N, D = x.shape
    E = edge_attr.shape[0]
    src = edge_index[0]
    dst = edge_index[1]

    feat = _edge_feat(edge_attr, We.T, be.reshape(1, -1))
    agg2 = _make_sc_scatter(N, D, E)(x, src, dst, feat)
    out = _combine_mlp(
        x, agg2, epsilon,
        W1.T, b1.reshape(1, -1), g1.reshape(1, -1), bt1.reshape(1, -1),
        W2.T, b2.reshape(1, -1), g2.reshape(1, -1), bt2.reshape(1, -1))
    return out
